# initial kernel scaffold (unmeasured)
import jax
import jax.numpy as jnp
from jax import lax
from jax.experimental import pallas as pl
from jax.experimental.pallas import tpu as pltpu

N_X = 2
E_GLB = 16
CAP = 320
N_FBLK = 4


def _mm3(a, b):
    ah = a.astype(jnp.bfloat16).astype(jnp.float32)
    al = a - ah
    bh = b.astype(jnp.bfloat16).astype(jnp.float32)
    bl = b - bh
    return jnp.dot(ah, bh) + (jnp.dot(ah, bl) + jnp.dot(al, bh))


def _cumsum0(v, n):
    shift = 1
    while shift < n:
        z = jnp.zeros((shift, 1), v.dtype)
        v = v + jnp.concatenate([z, v[:-shift]], axis=0)
        shift *= 2
    return v


def kernel(x, router, W1, W2):
    t_loc, d = x.shape
    e_loc, _, f = W1.shape
    t_glb = N_X * t_loc
    fb = f // N_FBLK

    def body(x_ref, r_ref, w1_ref, w2_ref, out_ref,
             xall, rall, meta, part, rbuf, s_ref, xg, yacc, sems):
        e = pl.program_id(0)
        blk = pl.program_id(1)
        my_x = lax.axis_index("x")
        my_y = lax.axis_index("y")
        peer = (1 - my_x, my_y)

        @pl.when((e == 0) & (blk == 0))
        def _dispatch():
            bar = pltpu.get_barrier_semaphore()
            pl.semaphore_signal(bar, inc=1, device_id=peer,
                                device_id_type=pl.DeviceIdType.MESH)
            pl.semaphore_wait(bar, 1)

            xall[pl.ds(my_x * t_loc, t_loc), :] = x_ref[...]
            rall[my_x] = r_ref[...]
            rd_x = pltpu.make_async_remote_copy(
                src_ref=xall.at[pl.ds(my_x * t_loc, t_loc), :],
                dst_ref=xall.at[pl.ds(my_x * t_loc, t_loc), :],
                send_sem=sems.at[0], recv_sem=sems.at[1],
                device_id=peer, device_id_type=pl.DeviceIdType.MESH,
            )
            rd_r = pltpu.make_async_remote_copy(
                src_ref=rall.at[my_x],
                dst_ref=rall.at[my_x],
                send_sem=sems.at[2], recv_sem=sems.at[3],
                device_id=peer, device_id_type=pl.DeviceIdType.MESH,
            )
            rd_x.start()
            rd_r.start()
            rd_x.wait()
            rd_r.wait()

            xa = xall[...]
            g = jnp.concatenate(
                [_mm3(xa, rall[0]), _mm3(xa, rall[1])], axis=1)
            colf = lax.broadcasted_iota(jnp.float32, g.shape, 1)
            v1 = jnp.max(g, axis=1, keepdims=True)
            i1 = jnp.min(jnp.where(g == v1, colf, 1e9), axis=1, keepdims=True)
            gm = jnp.where(colf == i1, -1e30, g)
            v2 = jnp.max(gm, axis=1, keepdims=True)
            i2 = jnp.min(jnp.where(gm == v2, colf, 1e9), axis=1, keepdims=True)
            ew = jnp.exp(v2 - v1)
            meta[0] = i1
            meta[1] = i2
            meta[2] = 1.0 / (1.0 + ew)
            meta[3] = ew / (1.0 + ew)

        ge_f = (my_x * e_loc + e).astype(jnp.float32)
        a1 = (meta[0] == ge_f).astype(jnp.float32)
        a2 = (meta[1] == ge_f).astype(jnp.float32)

        @pl.when(blk == 0)
        def _gather():
            m = a1 + a2
            slot = _cumsum0(m, t_glb) - 1.0
            ci = lax.broadcasted_iota(jnp.float32, (t_glb, CAP), 1)
            s_ref[...] = jnp.where((ci == slot) & (m > 0.0), 1.0, 0.0)
            xg[...] = lax.dot_general(
                s_ref[...], xall[...], (((0,), (0,)), ((), ())))

        h = jnp.maximum(jnp.dot(xg[...], w1_ref[0]), 0.0)
        yblk = jnp.dot(h, w2_ref[0])

        @pl.when(blk == 0)
        def _():
            yacc[...] = yblk

        @pl.when(blk > 0)
        def _():
            yacc[...] = yacc[...] + yblk

        @pl.when(blk == N_FBLK - 1)
        def _scatter():
            wsel = meta[2] * a1 + meta[3] * a2
            contrib = jnp.dot(s_ref[...] * wsel, yacc[...])

            @pl.when(e == 0)
            def _():
                part[...] = contrib

            @pl.when(e > 0)
            def _():
                part[...] = part[...] + contrib

        @pl.when((e == e_loc - 1) & (blk == N_FBLK - 1))
        def _combine():
            rd_c = pltpu.make_async_remote_copy(
                src_ref=part.at[pl.ds((1 - my_x) * t_loc, t_loc), :],
                dst_ref=rbuf,
                send_sem=sems.at[4], recv_sem=sems.at[5],
                device_id=peer, device_id_type=pl.DeviceIdType.MESH,
            )
            rd_c.start()
            rd_c.wait()
            out_ref[...] = part[pl.ds(my_x * t_loc, t_loc), :] + rbuf[...]

    return pl.pallas_call(
        body,
        grid=(e_loc, N_FBLK),
        out_shape=jax.ShapeDtypeStruct((t_loc, d), jnp.float32),
        in_specs=[
            pl.BlockSpec((t_loc, d), lambda e, b: (0, 0)),
            pl.BlockSpec((d, e_loc), lambda e, b: (0, 0)),
            pl.BlockSpec((1, d, fb), lambda e, b: (e, 0, b)),
            pl.BlockSpec((1, fb, d), lambda e, b: (e, b, 0)),
        ],
        out_specs=pl.BlockSpec((t_loc, d), lambda e, b: (0, 0)),
        scratch_shapes=[
            pltpu.VMEM((t_glb, d), jnp.float32),
            pltpu.VMEM((N_X, d, e_loc), jnp.float32),
            pltpu.VMEM((4, t_glb, 1), jnp.float32),
            pltpu.VMEM((t_glb, d), jnp.float32),
            pltpu.VMEM((t_loc, d), jnp.float32),
            pltpu.VMEM((t_glb, CAP), jnp.float32),
            pltpu.VMEM((CAP, d), jnp.float32),
            pltpu.VMEM((CAP, d), jnp.float32),
            pltpu.SemaphoreType.DMA((6,)),
        ],
        compiler_params=pltpu.CompilerParams(collective_id=0),
    )(x, router, W1, W2)


# baseline (device time: 248846 ns/iter reference)
import jax
import jax.numpy as jnp
from jax import lax
from jax.experimental import pallas as pl
from jax.experimental.pallas import tpu as pltpu

N_X = 2
E_GLB = 16
CAP = 320
N_FBLK = 4


def _mm3(a, b):
    ah = a.astype(jnp.bfloat16).astype(jnp.float32)
    al = a - ah
    bh = b.astype(jnp.bfloat16).astype(jnp.float32)
    bl = b - bh
    return jnp.dot(ah, bh) + (jnp.dot(ah, bl) + jnp.dot(al, bh))


def _cumsum0(v, n):
    shift = 1
    while shift < n:
        z = jnp.zeros((shift, 1), v.dtype)
        v = v + jnp.concatenate([z, v[:-shift]], axis=0)
        shift *= 2
    return v


def _dotT(a, b):
    return lax.dot_general(a, b, (((0,), (0,)), ((), ())))


def kernel(x, router, W1, W2):
    t_loc, d = x.shape
    e_loc, _, f = W1.shape
    t_glb = N_X * t_loc
    fb = f // N_FBLK

    def body(x_ref, r_ref, w1_ref, w2_ref, out_ref,
             xrem, rall, meta, ploc, prem, rbuf, s_ref, xg, yacc, sems):
        e = pl.program_id(0)
        blk = pl.program_id(1)
        my_x = lax.axis_index("x")
        my_y = lax.axis_index("y")
        peer = (1 - my_x, my_y)

        @pl.when((e == 0) & (blk == 0))
        def _dispatch():
            bar = pltpu.get_barrier_semaphore()
            pl.semaphore_signal(bar, inc=1, device_id=peer,
                                device_id_type=pl.DeviceIdType.MESH)
            pl.semaphore_wait(bar, 1)

            rall[my_x] = r_ref[...]
            rd_x = pltpu.make_async_remote_copy(
                src_ref=x_ref, dst_ref=xrem,
                send_sem=sems.at[0], recv_sem=sems.at[1],
                device_id=peer, device_id_type=pl.DeviceIdType.MESH,
            )
            rd_r = pltpu.make_async_remote_copy(
                src_ref=rall.at[my_x], dst_ref=rall.at[my_x],
                send_sem=sems.at[2], recv_sem=sems.at[3],
                device_id=peer, device_id_type=pl.DeviceIdType.MESH,
            )
            rd_x.start()
            rd_r.start()
            rd_x.wait()
            rd_r.wait()

            ra = rall[0]
            rb = rall[1]
            g = jnp.concatenate(
                [jnp.concatenate([_mm3(x_ref[...], ra),
                                  _mm3(x_ref[...], rb)], axis=1),
                 jnp.concatenate([_mm3(xrem[...], ra),
                                  _mm3(xrem[...], rb)], axis=1)], axis=0)
            colf = lax.broadcasted_iota(jnp.int32, g.shape, 1).astype(
                jnp.float32)
            v1 = jnp.max(g, axis=1, keepdims=True)
            i1 = jnp.min(jnp.where(g == v1, colf, 1e9), axis=1, keepdims=True)
            gm = jnp.where(colf == i1, -1e30, g)
            v2 = jnp.max(gm, axis=1, keepdims=True)
            i2 = jnp.min(jnp.where(gm == v2, colf, 1e9), axis=1, keepdims=True)
            ew = jnp.exp(v2 - v1)
            w1c = 1.0 / (1.0 + ew)
            meta[...] = jnp.concatenate([i1, i2, w1c, 1.0 - w1c], axis=1)

        ge_f = (my_x * e_loc + e).astype(jnp.float32)
        a1 = (meta[:, 0:1] == ge_f).astype(jnp.float32)
        a2 = (meta[:, 1:2] == ge_f).astype(jnp.float32)

        @pl.when(blk == 0)
        def _gather():
            m = a1 + a2
            slot = _cumsum0(m, t_glb) - 1.0
            ci = lax.broadcasted_iota(jnp.int32, (t_glb, CAP), 1).astype(
                jnp.float32)
            s_ref[...] = jnp.where((ci == slot) & (m > 0.0), 1.0, 0.0)
            xg[...] = (_dotT(s_ref[:t_loc], x_ref[...])
                       + _dotT(s_ref[t_loc:], xrem[...]))

        h = jnp.maximum(jnp.dot(xg[...], w1_ref[0]), 0.0)
        yblk = jnp.dot(h, w2_ref[0])

        @pl.when(blk == 0)
        def _():
            yacc[...] = yblk

        @pl.when(blk > 0)
        def _():
            yacc[...] = yacc[...] + yblk

        @pl.when(blk == N_FBLK - 1)
        def _scatter():
            wsel = meta[:, 2:3] * a1 + meta[:, 3:4] * a2
            sw = s_ref[...] * wsel
            cl = jnp.dot(sw[:t_loc], yacc[...])
            cr = jnp.dot(sw[t_loc:], yacc[...])

            @pl.when(e == 0)
            def _():
                ploc[...] = cl
                prem[...] = cr

            @pl.when(e > 0)
            def _():
                ploc[...] = ploc[...] + cl
                prem[...] = prem[...] + cr

        @pl.when((e == e_loc - 1) & (blk == N_FBLK - 1))
        def _combine():
            rd_c = pltpu.make_async_remote_copy(
                src_ref=prem, dst_ref=rbuf,
                send_sem=sems.at[4], recv_sem=sems.at[5],
                device_id=peer, device_id_type=pl.DeviceIdType.MESH,
            )
            rd_c.start()
            rd_c.wait()
            out_ref[...] = ploc[...] + rbuf[...]

    return pl.pallas_call(
        body,
        grid=(e_loc, N_FBLK),
        out_shape=jax.ShapeDtypeStruct((t_loc, d), jnp.float32),
        in_specs=[
            pl.BlockSpec((t_loc, d), lambda e, b: (0, 0)),
            pl.BlockSpec((d, e_loc), lambda e, b: (0, 0)),
            pl.BlockSpec((1, d, fb), lambda e, b: (e, 0, b)),
            pl.BlockSpec((1, fb, d), lambda e, b: (e, b, 0)),
        ],
        out_specs=pl.BlockSpec((t_loc, d), lambda e, b: (0, 0)),
        scratch_shapes=[
            pltpu.VMEM((t_loc, d), jnp.float32),
            pltpu.VMEM((N_X, d, e_loc), jnp.float32),
            pltpu.VMEM((t_glb, 4), jnp.float32),
            pltpu.VMEM((t_loc, d), jnp.float32),
            pltpu.VMEM((t_loc, d), jnp.float32),
            pltpu.VMEM((t_loc, d), jnp.float32),
            pltpu.VMEM((t_glb, CAP), jnp.float32),
            pltpu.VMEM((CAP, d), jnp.float32),
            pltpu.VMEM((CAP, d), jnp.float32),
            pltpu.SemaphoreType.DMA((6,)),
        ],
        compiler_params=pltpu.CompilerParams(
            collective_id=0, vmem_limit_bytes=60 * 1024 * 1024),
    )(x, router, W1, W2)
